# Initial kernel scaffold; baseline (speedup 1.0000x reference)
import jax
import jax.numpy as jnp
from jax.experimental import pallas as pl


def kernel(dyn_seq, static, edge_index, W1, b1, W2, b2, Wih, Whh, bih, bhh, Wm1, bm1, Wm2, bm2):
    raise NotImplementedError("write your pallas kernel here")


# speedup vs baseline: 1.4125x; 1.4125x over previous
import functools

import jax
import jax.numpy as jnp
from jax import lax
from jax.experimental import pallas as pl
from jax.experimental.pallas import tpu as pltpu

T = 12
GH = 32
RH = 64

_INTERPRET = False


def _ka_body(static_ref, dyn_ref, dinv_ref, w1s_ref, u_ref):
    dinv = dinv_ref[...]
    s = jnp.dot(static_ref[...], w1s_ref[...], preferred_element_type=jnp.float32)
    u_ref[0] = dinv * s
    d = dinv * dyn_ref[...]
    u_ref[1] = jnp.pad(d, ((0, 0), (0, GH - T)))


def _build_u(static, dyn_nt, dinv, W1, n, blk):
    grid = n // blk
    return pl.pallas_call(
        _ka_body,
        grid=(grid,),
        in_specs=[
            pl.BlockSpec((blk, static.shape[1]), lambda i: (i, 0)),
            pl.BlockSpec((blk, T), lambda i: (i, 0)),
            pl.BlockSpec((blk, 1), lambda i: (i, 0)),
            pl.BlockSpec(W1[1:].shape, lambda i: (0, 0)),
        ],
        out_specs=pl.BlockSpec((2, blk, GH), lambda i: (0, i, 0)),
        out_shape=jax.ShapeDtypeStruct((2, n, GH), jnp.float32),
        interpret=_INTERPRET,
    )(static, dyn_nt, dinv, W1[1:])


def _kb_body(acc_ref, dinv_ref, w1d_ref, b1_ref, h_ref):
    dinv = dinv_ref[...]
    ps = dinv * acc_ref[0] + b1_ref[...]
    pd = dinv * acc_ref[1]
    w1d = w1d_ref[...]
    for t in range(T):
        h1 = jax.nn.relu(pd[:, t : t + 1] * w1d + ps)
        h_ref[t] = dinv * h1


def _build_h1s(acc1, dinv, W1, b1, n, blk):
    grid = n // blk
    return pl.pallas_call(
        _kb_body,
        grid=(grid,),
        in_specs=[
            pl.BlockSpec((2, blk, GH), lambda i: (0, i, 0)),
            pl.BlockSpec((blk, 1), lambda i: (i, 0)),
            pl.BlockSpec((1, GH), lambda i: (0, 0)),
            pl.BlockSpec((1, GH), lambda i: (0, 0)),
        ],
        out_specs=pl.BlockSpec((T, blk, GH), lambda i: (0, i, 0)),
        out_shape=jax.ShapeDtypeStruct((T, n, GH), jnp.float32),
        interpret=_INTERPRET,
    )(acc1, dinv, W1[0].reshape(1, GH), b1.reshape(1, GH))


def _kc_body(acc_ref, dinv_ref, W2_ref, b2_ref, Wih_ref, Whh_ref, bih_ref,
             bhh_ref, Wm1_ref, bm1_ref, Wm2_ref, bm2_ref, out_ref):
    dinv = dinv_ref[...]
    nb = dinv.shape[0]
    W2 = W2_ref[...]
    b2 = b2_ref[...]
    Wih = Wih_ref[...]
    Whh = Whh_ref[...]
    bih = bih_ref[...]
    bhh = bhh_ref[...]
    h = jnp.zeros((nb, RH), jnp.float32)
    for t in range(T):
        a = dinv * acc_ref[t]
        x = jax.nn.relu(jnp.dot(a, W2, preferred_element_type=jnp.float32) + b2)
        gi = jnp.dot(x, Wih, preferred_element_type=jnp.float32) + bih
        gh = jnp.dot(h, Whh, preferred_element_type=jnp.float32) + bhh
        r = jax.nn.sigmoid(gi[:, :RH] + gh[:, :RH])
        z = jax.nn.sigmoid(gi[:, RH : 2 * RH] + gh[:, RH : 2 * RH])
        nn = jnp.tanh(gi[:, 2 * RH :] + r * gh[:, 2 * RH :])
        h = (1.0 - z) * nn + z * h
    m = jax.nn.relu(jnp.dot(h, Wm1_ref[...], preferred_element_type=jnp.float32)
                    + bm1_ref[...])
    out_ref[...] = (jnp.dot(m, Wm2_ref[...], preferred_element_type=jnp.float32)
                    + bm2_ref[...])


def _head(acc2, dinv, W2, b2, Wih, Whh, bih, bhh, Wm1, bm1, Wm2, bm2, n, blk):
    grid = n // blk
    full = lambda a: pl.BlockSpec(a.shape, lambda i: tuple(0 for _ in a.shape))
    args = (W2, b2.reshape(1, GH), Wih, Whh, bih.reshape(1, 3 * RH),
            bhh.reshape(1, 3 * RH), Wm1, bm1.reshape(1, RH), Wm2,
            bm2.reshape(1, 1))
    return pl.pallas_call(
        _kc_body,
        grid=(grid,),
        in_specs=[
            pl.BlockSpec((T, blk, GH), lambda i: (0, i, 0)),
            pl.BlockSpec((blk, 1), lambda i: (i, 0)),
        ] + [full(a) for a in args],
        out_specs=pl.BlockSpec((blk, 1), lambda i: (i, 0)),
        out_shape=jax.ShapeDtypeStruct((n, 1), jnp.float32),
        interpret=_INTERPRET,
    )(acc2, dinv, *args)


def kernel(dyn_seq, static, edge_index, W1, b1, W2, b2, Wih, Whh, bih, bhh,
           Wm1, bm1, Wm2, bm2):
    n = static.shape[0]
    blk = 2000
    src, dst = edge_index[0], edge_index[1]

    deg = 1.0 + jnp.zeros((n,), jnp.float32).at[dst].add(1.0)
    dinv = lax.rsqrt(deg).reshape(n, 1)

    dyn_nt = dyn_seq[:, :, 0].T
    u = _build_u(static, dyn_nt, dinv, W1, n, blk)
    acc1 = u.at[:, dst, :].add(u[:, src, :])
    h1s = _build_h1s(acc1, dinv, W1, b1, n, blk)
    acc2 = h1s.at[:, dst, :].add(h1s[:, src, :])
    return _head(acc2, dinv, W2, b2, Wih, Whh, bih, bhh, Wm1, bm1, Wm2, bm2,
                 n, blk)[:, 0]
